# Initial kernel scaffold; baseline (speedup 1.0000x reference)
import jax
import jax.numpy as jnp
from jax.experimental import pallas as pl


def kernel(x, Wr, W1, b1, W2, b2):
    raise NotImplementedError("write your pallas kernel here")


# speedup vs baseline: 1.2285x; 1.2285x over previous
import functools

import jax
import jax.numpy as jnp
from jax import lax
from jax.experimental import pallas as pl
from jax.experimental.pallas import tpu as pltpu

B, T, D = 2, 2048, 1024
DFF = 4096
E = 8
K = 2
N = B * T
A = N * K
BLK = 256
NB = A // BLK + E
NP = NB * BLK
DFC = 2048
DC = DFF // DFC


def _router_body(x_ref, wr_ref, out_ref):
    logits = jnp.dot(x_ref[...], wr_ref[...], preferred_element_type=jnp.float32)
    lane = lax.broadcasted_iota(jnp.int32, logits.shape, 1)
    neg = jnp.float32(-1e30)
    l0 = jnp.where(lane < E, logits, neg)
    m0 = jnp.max(l0, axis=1)
    i0 = jnp.min(jnp.where(l0 >= m0[:, None], lane, 127), axis=1)
    l1 = jnp.where(lane == i0[:, None], neg, l0)
    m1 = jnp.max(l1, axis=1)
    i1 = jnp.min(jnp.where(l1 >= m1[:, None], lane, 127), axis=1)
    w0 = 1.0 / (1.0 + jnp.exp(m1 - m0))
    w1 = 1.0 - w0
    out_ref[0:1, :] = i0.astype(jnp.float32)[None, :]
    out_ref[1:2, :] = i1.astype(jnp.float32)[None, :]
    out_ref[2:3, :] = w0[None, :]
    out_ref[3:4, :] = w1[None, :]


def _route(x_flat, Wr):
    wr_pad = jnp.pad(Wr, ((0, 0), (0, 128 - E)))
    r = pl.pallas_call(
        _router_body,
        out_shape=jax.ShapeDtypeStruct((8, N), jnp.float32),
    )(x_flat, wr_pad)
    i0 = r[0].astype(jnp.int32)
    i1 = r[1].astype(jnp.int32)
    return i0, i1, r[2], r[3]


def _ffn_body(be_ref, xs_ref, w1_ref, b1_ref, w2_ref, b2_ref, out_ref):
    c = pl.program_id(1)
    h = jnp.dot(xs_ref[...], w1_ref[0], preferred_element_type=jnp.float32)
    h = jnp.maximum(h + b1_ref[0], 0.0)
    o = jnp.dot(h, w2_ref[0], preferred_element_type=jnp.float32)

    @pl.when(c == 0)
    def _():
        out_ref[...] = o + b2_ref[0]

    @pl.when(c != 0)
    def _():
        out_ref[...] += o


def _expert_ffn(block_expert, xs, W1, b1, W2, b2):
    grid_spec = pltpu.PrefetchScalarGridSpec(
        num_scalar_prefetch=1,
        grid=(NB, DC),
        in_specs=[
            pl.BlockSpec((BLK, D), lambda i, c, be: (i, 0)),
            pl.BlockSpec((1, D, DFC), lambda i, c, be: (be[i], 0, c)),
            pl.BlockSpec((1, 1, DFC), lambda i, c, be: (be[i], 0, c)),
            pl.BlockSpec((1, DFC, D), lambda i, c, be: (be[i], c, 0)),
            pl.BlockSpec((1, 1, D), lambda i, c, be: (be[i], 0, 0)),
        ],
        out_specs=pl.BlockSpec((BLK, D), lambda i, c, be: (i, 0)),
    )
    return pl.pallas_call(
        _ffn_body,
        grid_spec=grid_spec,
        out_shape=jax.ShapeDtypeStruct((NP, D), jnp.float32),
        compiler_params=pltpu.CompilerParams(
            dimension_semantics=("arbitrary", "arbitrary"),
        ),
    )(block_expert, xs, W1, b1.reshape(E, 1, DFF), W2, b2.reshape(E, 1, D))


def kernel(x, Wr, W1, b1, W2, b2):
    x_flat = x.reshape(N, D)
    i0, i1, w0, w1 = _route(x_flat, Wr)

    eid_all = jnp.concatenate([i0, i1])
    counts = jnp.zeros((E,), jnp.int32).at[eid_all].add(1)
    pc = (counts + (BLK - 1)) // BLK * BLK
    base = jnp.concatenate([jnp.zeros((1,), jnp.int32), jnp.cumsum(pc)])
    off = jnp.concatenate([jnp.zeros((1,), jnp.int32), jnp.cumsum(counts)])
    order = jnp.argsort(eid_all, stable=True)
    g = eid_all[order]
    target = jnp.arange(A, dtype=jnp.int32) + (base[g] - off[g])
    pos_all = jnp.zeros((A,), jnp.int32).at[order].set(target)
    posA, posB = pos_all[:N], pos_all[N:]

    tok_all = jnp.concatenate([jnp.arange(N, dtype=jnp.int32)] * 2)
    gidx = jnp.zeros((NP,), jnp.int32).at[pos_all].set(tok_all)
    xs = x_flat[gidx]

    block_expert = jnp.clip(
        jnp.searchsorted(base, jnp.arange(NB, dtype=jnp.int32) * BLK,
                         side="right").astype(jnp.int32) - 1, 0, E - 1)

    os_rows = _expert_ffn(block_expert, xs, W1, b1, W2, b2)

    y = w0[:, None] * os_rows[posA] + w1[:, None] * os_rows[posB]

    usage_counts = counts.astype(x.dtype)
    usage_fraction = usage_counts / jnp.float32(A)
    zero = jnp.zeros((), dtype=x.dtype)
    return (y.reshape(B, T, D), usage_counts, usage_fraction, zero)


# speedup vs baseline: 1.4868x; 1.2102x over previous
import functools

import jax
import jax.numpy as jnp
from jax import lax
from jax.experimental import pallas as pl
from jax.experimental.pallas import tpu as pltpu
from jax.experimental.pallas import tpu_sc as plsc

B, T, D = 2, 2048, 1024
DFF = 4096
E = 8
K = 2
N = B * T
A = N * K
BLK = 256
NB = A // BLK + E
NP = NB * BLK
DFC = 2048
DC = DFF // DFC

NW = 32
TPW = N // NW
CH = TPW // 16

_mesh = functools.partial(
    plsc.VectorSubcoreMesh, core_axis_name="c", subcore_axis_name="s")
_SC_PARAMS = pltpu.CompilerParams(needs_layout_passes=False)


def _wid():
    return lax.axis_index("s") * 2 + lax.axis_index("c")


def _router_body(x_ref, wr_ref, out_ref):
    logits = jnp.dot(x_ref[...], wr_ref[...], preferred_element_type=jnp.float32)
    lane = lax.broadcasted_iota(jnp.int32, logits.shape, 1)
    neg = jnp.float32(-1e30)
    l0 = jnp.where(lane < E, logits, neg)
    m0 = jnp.max(l0, axis=1)
    i0 = jnp.min(jnp.where(l0 >= m0[:, None], lane, 127), axis=1)
    l1 = jnp.where(lane == i0[:, None], neg, l0)
    m1 = jnp.max(l1, axis=1)
    i1 = jnp.min(jnp.where(l1 >= m1[:, None], lane, 127), axis=1)
    w0 = 1.0 / (1.0 + jnp.exp(m1 - m0))
    w1 = 1.0 - w0
    out_ref[0:1, :] = i0.astype(jnp.float32)[None, :]
    out_ref[1:2, :] = i1.astype(jnp.float32)[None, :]
    out_ref[2:3, :] = w0[None, :]
    out_ref[3:4, :] = w1[None, :]


def _route(x_flat, Wr):
    wr_pad = jnp.pad(Wr, ((0, 0), (0, 128 - E)))
    r = pl.pallas_call(
        _router_body,
        out_shape=jax.ShapeDtypeStruct((8, N), jnp.float32),
    )(x_flat, wr_pad)
    i0 = r[0].astype(jnp.int32)
    i1 = r[1].astype(jnp.int32)
    return i0, i1, r[2], r[3]


def _count_body(i0_hbm, i1_hbm, lrA_hbm, lrB_hbm, cnt_hbm, eid_v, lr_v, cnt_ref):
    w = _wid()
    lane = lax.iota(jnp.int32, 16)
    cnt_ref[...] = jnp.zeros((16,), jnp.int32)

    for half, (src, dst) in enumerate(((i0_hbm, lrA_hbm), (i1_hbm, lrB_hbm))):
        pltpu.sync_copy(src.at[pl.ds(w * TPW, TPW)], eid_v)

        def chunk(j, _):
            v = eid_v[pl.ds(j * 16, 16)]
            prior = plsc.load_gather(cnt_ref, [v])
            within = jnp.zeros((16,), jnp.int32)
            cnt_vec = cnt_ref[...]
            for e in range(E):
                m = v == e
                cs = plsc.cumsum(jnp.where(m, 1, 0).astype(jnp.int32))
                within = jnp.where(m, cs - 1, within)
                tot_e = jnp.sum(jnp.where(m, 1, 0).astype(jnp.int32))
                cnt_vec = jnp.where(lane == e, cnt_vec + tot_e, cnt_vec)
            cnt_ref[...] = cnt_vec
            lr_v[pl.ds(j * 16, 16)] = prior + within
            return 0

        lax.fori_loop(0, CH, chunk, 0)
        pltpu.sync_copy(lr_v, dst.at[pl.ds(w * TPW, TPW)])

    pltpu.sync_copy(cnt_ref, cnt_hbm.at[w])


def _sc_count(i0, i1):
    f = pl.kernel(
        _count_body,
        out_type=[
            jax.ShapeDtypeStruct((N,), jnp.int32),
            jax.ShapeDtypeStruct((N,), jnp.int32),
            jax.ShapeDtypeStruct((NW, 16), jnp.int32),
        ],
        mesh=_mesh(),
        compiler_params=_SC_PARAMS,
        scratch_types=[
            pltpu.VMEM((TPW,), jnp.int32),
            pltpu.VMEM((TPW,), jnp.int32),
            pltpu.VMEM((16,), jnp.int32),
        ],
    )
    return f(i0, i1)


def _scatter_body(x_hbm, i0_hbm, i1_hbm, lrA_hbm, lrB_hbm, off_hbm,
                  xs_hbm, posA_hbm, posB_hbm,
                  eid_v, lr_v, posA_v, posB_v, off_ref, xbuf, semA, semB):
    w = _wid()
    pltpu.sync_copy(off_hbm.at[w], off_ref)

    for src, lr_src, pos_v, pos_dst in (
            (i0_hbm, lrA_hbm, posA_v, posA_hbm),
            (i1_hbm, lrB_hbm, posB_v, posB_hbm)):
        pltpu.sync_copy(src.at[pl.ds(w * TPW, TPW)], eid_v)
        pltpu.sync_copy(lr_src.at[pl.ds(w * TPW, TPW)], lr_v)

        def chunk(j, _):
            v = eid_v[pl.ds(j * 16, 16)]
            pos_v[pl.ds(j * 16, 16)] = (
                lr_v[pl.ds(j * 16, 16)] + plsc.load_gather(off_ref, [v]))
            return 0

        lax.fori_loop(0, CH, chunk, 0)
        pltpu.sync_copy(pos_v, pos_dst.at[pl.ds(w * TPW, TPW)])

    for j in range(CH):
        pltpu.sync_copy(x_hbm.at[pl.ds(w * TPW + j * 16, 16)], xbuf)
        a = posA_v[pl.ds(j * 16, 16)]
        b = posB_v[pl.ds(j * 16, 16)]
        cpA = pltpu.async_copy(xbuf, xs_hbm.at[a], semA)
        cpB = pltpu.async_copy(xbuf, xs_hbm.at[b], semB)
        cpA.wait()
        cpB.wait()


def _sc_scatter(x_flat, i0, i1, lrA, lrB, off):
    f = pl.kernel(
        _scatter_body,
        out_type=[
            jax.ShapeDtypeStruct((NP, D), jnp.float32),
            jax.ShapeDtypeStruct((N,), jnp.int32),
            jax.ShapeDtypeStruct((N,), jnp.int32),
        ],
        mesh=_mesh(),
        compiler_params=_SC_PARAMS,
        scratch_types=[
            pltpu.VMEM((TPW,), jnp.int32),
            pltpu.VMEM((TPW,), jnp.int32),
            pltpu.VMEM((TPW,), jnp.int32),
            pltpu.VMEM((TPW,), jnp.int32),
            pltpu.VMEM((16,), jnp.int32),
            pltpu.VMEM((16, D), jnp.float32),
            pltpu.SemaphoreType.DMA,
            pltpu.SemaphoreType.DMA,
        ],
    )
    return f(x_flat, i0, i1, lrA, lrB, off)


def _combine_body(os_hbm, posA_hbm, posB_hbm, w0_hbm, w1_hbm, y_hbm,
                  posA_v, posB_v, w0_v, w1_v, bufA, bufB, ybuf, semA, semB):
    w = _wid()
    lane = lax.iota(jnp.int32, 16)
    pltpu.sync_copy(posA_hbm.at[pl.ds(w * TPW, TPW)], posA_v)
    pltpu.sync_copy(posB_hbm.at[pl.ds(w * TPW, TPW)], posB_v)
    pltpu.sync_copy(w0_hbm.at[pl.ds(w * TPW, TPW)], w0_v)
    pltpu.sync_copy(w1_hbm.at[pl.ds(w * TPW, TPW)], w1_v)

    for j in range(CH):
        a = posA_v[pl.ds(j * 16, 16)]
        b = posB_v[pl.ds(j * 16, 16)]
        cpA = pltpu.async_copy(os_hbm.at[a], bufA, semA)
        cpB = pltpu.async_copy(os_hbm.at[b], bufB, semB)
        cpA.wait()
        cpB.wait()
        w0c = w0_v[pl.ds(j * 16, 16)]
        w1c = w1_v[pl.ds(j * 16, 16)]

        def row(r, _):
            wa = jnp.sum(jnp.where(lane == r, w0c, 0.0))
            wb = jnp.sum(jnp.where(lane == r, w1c, 0.0))

            def col(c, _):
                ybuf[r, pl.ds(c * 16, 16)] = (
                    wa * bufA[r, pl.ds(c * 16, 16)]
                    + wb * bufB[r, pl.ds(c * 16, 16)])
                return 0

            lax.fori_loop(0, D // 16, col, 0)
            return 0

        lax.fori_loop(0, 16, row, 0)
        pltpu.sync_copy(ybuf, y_hbm.at[pl.ds(w * TPW + j * 16, 16)])


def _sc_combine(os_rows, posA, posB, w0, w1):
    f = pl.kernel(
        _combine_body,
        out_type=jax.ShapeDtypeStruct((N, D), jnp.float32),
        mesh=_mesh(),
        compiler_params=_SC_PARAMS,
        scratch_types=[
            pltpu.VMEM((TPW,), jnp.int32),
            pltpu.VMEM((TPW,), jnp.int32),
            pltpu.VMEM((TPW,), jnp.float32),
            pltpu.VMEM((TPW,), jnp.float32),
            pltpu.VMEM((16, D), jnp.float32),
            pltpu.VMEM((16, D), jnp.float32),
            pltpu.VMEM((16, D), jnp.float32),
            pltpu.SemaphoreType.DMA,
            pltpu.SemaphoreType.DMA,
        ],
    )
    return f(os_rows, posA, posB, w0, w1)


def _ffn_body(be_ref, xs_ref, w1_ref, b1_ref, w2_ref, b2_ref, out_ref):
    c = pl.program_id(1)
    h = jnp.dot(xs_ref[...], w1_ref[0], preferred_element_type=jnp.float32)
    h = jnp.maximum(h + b1_ref[0], 0.0)
    o = jnp.dot(h, w2_ref[0], preferred_element_type=jnp.float32)

    @pl.when(c == 0)
    def _():
        out_ref[...] = o + b2_ref[0]

    @pl.when(c != 0)
    def _():
        out_ref[...] += o


def _expert_ffn(block_expert, xs, W1, b1, W2, b2):
    grid_spec = pltpu.PrefetchScalarGridSpec(
        num_scalar_prefetch=1,
        grid=(NB, DC),
        in_specs=[
            pl.BlockSpec((BLK, D), lambda i, c, be: (i, 0)),
            pl.BlockSpec((1, D, DFC), lambda i, c, be: (be[i], 0, c)),
            pl.BlockSpec((1, 1, DFC), lambda i, c, be: (be[i], 0, c)),
            pl.BlockSpec((1, DFC, D), lambda i, c, be: (be[i], c, 0)),
            pl.BlockSpec((1, 1, D), lambda i, c, be: (be[i], 0, 0)),
        ],
        out_specs=pl.BlockSpec((BLK, D), lambda i, c, be: (i, 0)),
    )
    return pl.pallas_call(
        _ffn_body,
        grid_spec=grid_spec,
        out_shape=jax.ShapeDtypeStruct((NP, D), jnp.float32),
        compiler_params=pltpu.CompilerParams(
            dimension_semantics=("arbitrary", "arbitrary"),
        ),
    )(block_expert, xs, W1, b1.reshape(E, 1, DFF), W2, b2.reshape(E, 1, D))


def kernel(x, Wr, W1, b1, W2, b2):
    x_flat = x.reshape(N, D)
    i0, i1, w0, w1 = _route(x_flat, Wr)

    lrA, lrB, cnt_wt = _sc_count(i0, i1)

    cnt8 = cnt_wt[:, :E]
    tot = cnt8.sum(axis=0)
    pc = (tot + (BLK - 1)) // BLK * BLK
    base = jnp.concatenate([jnp.zeros((1,), jnp.int32), jnp.cumsum(pc)])
    pre = jnp.cumsum(cnt8, axis=0) - cnt8
    off = base[:E][None, :] + pre
    off16 = jnp.pad(off, ((0, 0), (0, 8)))

    block_expert = jnp.clip(
        jnp.searchsorted(base, jnp.arange(NB, dtype=jnp.int32) * BLK,
                         side="right").astype(jnp.int32) - 1, 0, E - 1)

    xs, posA, posB = _sc_scatter(x_flat, i0, i1, lrA, lrB, off16)

    os_rows = _expert_ffn(block_expert, xs, W1, b1, W2, b2)

    y = _sc_combine(os_rows, posA, posB, w0, w1)

    usage_counts = tot.astype(x.dtype)
    usage_fraction = usage_counts / jnp.float32(A)
    zero = jnp.zeros((), dtype=x.dtype)
    return (y.reshape(B, T, D), usage_counts, usage_fraction, zero)


# speedup vs baseline: 1.9601x; 1.3183x over previous
import functools

import jax
import jax.numpy as jnp
from jax import lax
from jax.experimental import pallas as pl
from jax.experimental.pallas import tpu as pltpu
from jax.experimental.pallas import tpu_sc as plsc

B, T, D = 2, 2048, 1024
DFF = 4096
E = 8
K = 2
N = B * T
A = N * K
BLK = 256
NB = A // BLK + E
NP = NB * BLK
DFC = 2048
DC = DFF // DFC

NW = 32
TPW = N // NW
CH = TPW // 16

_mesh = functools.partial(
    plsc.VectorSubcoreMesh, core_axis_name="c", subcore_axis_name="s")
_SC_PARAMS = pltpu.CompilerParams(needs_layout_passes=False)


def _wid():
    return lax.axis_index("s") * 2 + lax.axis_index("c")


def _router_body(x_ref, wr_ref, out_ref):
    logits = jnp.dot(x_ref[...], wr_ref[...], preferred_element_type=jnp.float32)
    lane = lax.broadcasted_iota(jnp.int32, logits.shape, 1)
    neg = jnp.float32(-1e30)
    l0 = jnp.where(lane < E, logits, neg)
    m0 = jnp.max(l0, axis=1)
    i0 = jnp.min(jnp.where(l0 >= m0[:, None], lane, 127), axis=1)
    l1 = jnp.where(lane == i0[:, None], neg, l0)
    m1 = jnp.max(l1, axis=1)
    i1 = jnp.min(jnp.where(l1 >= m1[:, None], lane, 127), axis=1)
    w0 = 1.0 / (1.0 + jnp.exp(m1 - m0))
    w1 = 1.0 - w0
    out_ref[0:1, :] = i0.astype(jnp.float32)[None, :]
    out_ref[1:2, :] = i1.astype(jnp.float32)[None, :]
    out_ref[2:3, :] = w0[None, :]
    out_ref[3:4, :] = w1[None, :]


def _route(x_flat, Wr):
    wr_pad = jnp.pad(Wr, ((0, 0), (0, 128 - E)))
    r = pl.pallas_call(
        _router_body,
        out_shape=jax.ShapeDtypeStruct((8, N), jnp.float32),
    )(x_flat, wr_pad)
    i0 = r[0].astype(jnp.int32)
    i1 = r[1].astype(jnp.int32)
    return i0, i1, r[2], r[3]


def _count_body(i0_hbm, i1_hbm, lrA_hbm, lrB_hbm, cnt_hbm, eid_v, lr_v, cnt_ref):
    w = _wid()
    lane = lax.iota(jnp.int32, 16)
    cnt_ref[...] = jnp.zeros((16,), jnp.int32)

    for half, (src, dst) in enumerate(((i0_hbm, lrA_hbm), (i1_hbm, lrB_hbm))):
        pltpu.sync_copy(src.at[pl.ds(w * TPW, TPW)], eid_v)

        def chunk(j, _):
            v = eid_v[pl.ds(j * 16, 16)]
            prior = plsc.load_gather(cnt_ref, [v])
            within = jnp.zeros((16,), jnp.int32)
            cnt_vec = cnt_ref[...]
            for e in range(E):
                m = v == e
                cs = plsc.cumsum(jnp.where(m, 1, 0).astype(jnp.int32))
                within = jnp.where(m, cs - 1, within)
                tot_e = jnp.sum(jnp.where(m, 1, 0).astype(jnp.int32))
                cnt_vec = jnp.where(lane == e, cnt_vec + tot_e, cnt_vec)
            cnt_ref[...] = cnt_vec
            lr_v[pl.ds(j * 16, 16)] = prior + within
            return 0

        lax.fori_loop(0, CH, chunk, 0)
        pltpu.sync_copy(lr_v, dst.at[pl.ds(w * TPW, TPW)])

    pltpu.sync_copy(cnt_ref, cnt_hbm.at[w])


def _sc_count(i0, i1):
    f = pl.kernel(
        _count_body,
        out_type=[
            jax.ShapeDtypeStruct((N,), jnp.int32),
            jax.ShapeDtypeStruct((N,), jnp.int32),
            jax.ShapeDtypeStruct((NW, 16), jnp.int32),
        ],
        mesh=_mesh(),
        compiler_params=_SC_PARAMS,
        scratch_types=[
            pltpu.VMEM((TPW,), jnp.int32),
            pltpu.VMEM((TPW,), jnp.int32),
            pltpu.VMEM((16,), jnp.int32),
        ],
    )
    return f(i0, i1)


def _scatter_body(x_hbm, i0_hbm, i1_hbm, lrA_hbm, lrB_hbm, off_hbm,
                  xs_hbm, posA_hbm, posB_hbm,
                  eid_v, lr_v, posA_v, posB_v, off_ref, xbuf, semA, semB):
    w = _wid()
    pltpu.sync_copy(off_hbm.at[w], off_ref)

    for src, lr_src, pos_v, pos_dst in (
            (i0_hbm, lrA_hbm, posA_v, posA_hbm),
            (i1_hbm, lrB_hbm, posB_v, posB_hbm)):
        pltpu.sync_copy(src.at[pl.ds(w * TPW, TPW)], eid_v)
        pltpu.sync_copy(lr_src.at[pl.ds(w * TPW, TPW)], lr_v)

        def chunk(j, _):
            v = eid_v[pl.ds(j * 16, 16)]
            pos_v[pl.ds(j * 16, 16)] = (
                lr_v[pl.ds(j * 16, 16)] + plsc.load_gather(off_ref, [v]))
            return 0

        lax.fori_loop(0, CH, chunk, 0)
        pltpu.sync_copy(pos_v, pos_dst.at[pl.ds(w * TPW, TPW)])

    for j in range(CH):
        pltpu.sync_copy(x_hbm.at[pl.ds(w * TPW + j * 16, 16)], xbuf)
        a = posA_v[pl.ds(j * 16, 16)]
        b = posB_v[pl.ds(j * 16, 16)]
        cpA = pltpu.async_copy(xbuf, xs_hbm.at[a], semA)
        cpB = pltpu.async_copy(xbuf, xs_hbm.at[b], semB)
        cpA.wait()
        cpB.wait()


def _sc_scatter(x_flat, i0, i1, lrA, lrB, off):
    f = pl.kernel(
        _scatter_body,
        out_type=[
            jax.ShapeDtypeStruct((NP, D), jnp.float32),
            jax.ShapeDtypeStruct((N,), jnp.int32),
            jax.ShapeDtypeStruct((N,), jnp.int32),
        ],
        mesh=_mesh(),
        compiler_params=_SC_PARAMS,
        scratch_types=[
            pltpu.VMEM((TPW,), jnp.int32),
            pltpu.VMEM((TPW,), jnp.int32),
            pltpu.VMEM((TPW,), jnp.int32),
            pltpu.VMEM((TPW,), jnp.int32),
            pltpu.VMEM((16,), jnp.int32),
            pltpu.VMEM((16, D), jnp.float32),
            pltpu.SemaphoreType.DMA,
            pltpu.SemaphoreType.DMA,
        ],
    )
    return f(x_flat, i0, i1, lrA, lrB, off)


def _combine_body(os_hbm, posA_hbm, posB_hbm, w0_hbm, w1_hbm, y_hbm,
                  posA_v, posB_v, w0_v, w1_v, bufA, bufB, bufA2, bufB2,
                  ybuf, semA, semB, semA2, semB2):
    w = _wid()
    lane = lax.iota(jnp.int32, 16)
    pltpu.sync_copy(posA_hbm.at[pl.ds(w * TPW, TPW)], posA_v)
    pltpu.sync_copy(posB_hbm.at[pl.ds(w * TPW, TPW)], posB_v)
    pltpu.sync_copy(w0_hbm.at[pl.ds(w * TPW, TPW)], w0_v)
    pltpu.sync_copy(w1_hbm.at[pl.ds(w * TPW, TPW)], w1_v)

    for j in range(CH):
        a = posA_v[pl.ds(j * 16, 16)]
        b = posB_v[pl.ds(j * 16, 16)]
        cpA = pltpu.async_copy(os_hbm.at[a], bufA, semA)
        cpB = pltpu.async_copy(os_hbm.at[b], bufB, semB)
        cpA2 = pltpu.async_copy(os_hbm.at[a + NP], bufA2, semA2)
        cpB2 = pltpu.async_copy(os_hbm.at[b + NP], bufB2, semB2)
        cpA.wait()
        cpB.wait()
        cpA2.wait()
        cpB2.wait()
        w0c = w0_v[pl.ds(j * 16, 16)]
        w1c = w1_v[pl.ds(j * 16, 16)]

        def row(r, _):
            wa = jnp.sum(jnp.where(lane == r, w0c, 0.0))
            wb = jnp.sum(jnp.where(lane == r, w1c, 0.0))

            def col(c, _):
                ybuf[r, pl.ds(c * 16, 16)] = (
                    wa * (bufA[r, pl.ds(c * 16, 16)]
                          + bufA2[r, pl.ds(c * 16, 16)])
                    + wb * (bufB[r, pl.ds(c * 16, 16)]
                            + bufB2[r, pl.ds(c * 16, 16)]))
                return 0

            lax.fori_loop(0, D // 16, col, 0)
            return 0

        lax.fori_loop(0, 16, row, 0)
        pltpu.sync_copy(ybuf, y_hbm.at[pl.ds(w * TPW + j * 16, 16)])


def _sc_combine(os_rows, posA, posB, w0, w1):
    f = pl.kernel(
        _combine_body,
        out_type=jax.ShapeDtypeStruct((N, D), jnp.float32),
        mesh=_mesh(),
        compiler_params=_SC_PARAMS,
        scratch_types=[
            pltpu.VMEM((TPW,), jnp.int32),
            pltpu.VMEM((TPW,), jnp.int32),
            pltpu.VMEM((TPW,), jnp.float32),
            pltpu.VMEM((TPW,), jnp.float32),
            pltpu.VMEM((16, D), jnp.float32),
            pltpu.VMEM((16, D), jnp.float32),
            pltpu.VMEM((16, D), jnp.float32),
            pltpu.VMEM((16, D), jnp.float32),
            pltpu.VMEM((16, D), jnp.float32),
            pltpu.SemaphoreType.DMA,
            pltpu.SemaphoreType.DMA,
            pltpu.SemaphoreType.DMA,
            pltpu.SemaphoreType.DMA,
        ],
    )
    return f(os_rows, posA, posB, w0, w1)


def _ffn_body(be_ref, xs_ref, w1_ref, b1_ref, w2_ref, b2_ref, out_ref):
    c = pl.program_id(0)
    h = jnp.dot(xs_ref[...], w1_ref[0], preferred_element_type=jnp.float32)
    h = jnp.maximum(h + b1_ref[0], 0.0)
    o = jnp.dot(h, w2_ref[0], preferred_element_type=jnp.float32)
    out_ref[...] = o + jnp.where(c == 0, 1.0, 0.0) * b2_ref[0]


def _expert_ffn(block_expert, xs, W1, b1, W2, b2):
    grid_spec = pltpu.PrefetchScalarGridSpec(
        num_scalar_prefetch=1,
        grid=(DC, NB),
        in_specs=[
            pl.BlockSpec((BLK, D), lambda c, i, be: (i, 0)),
            pl.BlockSpec((1, D, DFC), lambda c, i, be: (be[i], 0, c)),
            pl.BlockSpec((1, 1, DFC), lambda c, i, be: (be[i], 0, c)),
            pl.BlockSpec((1, DFC, D), lambda c, i, be: (be[i], c, 0)),
            pl.BlockSpec((1, 1, D), lambda c, i, be: (be[i], 0, 0)),
        ],
        out_specs=pl.BlockSpec((BLK, D), lambda c, i, be: (c * NB + i, 0)),
    )
    return pl.pallas_call(
        _ffn_body,
        grid_spec=grid_spec,
        out_shape=jax.ShapeDtypeStruct((DC * NP, D), jnp.float32),
        compiler_params=pltpu.CompilerParams(
            dimension_semantics=("arbitrary", "arbitrary"),
        ),
    )(block_expert, xs, W1, b1.reshape(E, 1, DFF), W2, b2.reshape(E, 1, D))


def kernel(x, Wr, W1, b1, W2, b2):
    x_flat = x.reshape(N, D)
    i0, i1, w0, w1 = _route(x_flat, Wr)

    lrA, lrB, cnt_wt = _sc_count(i0, i1)

    cnt8 = cnt_wt[:, :E]
    tot = cnt8.sum(axis=0)
    pc = (tot + (BLK - 1)) // BLK * BLK
    base = jnp.concatenate([jnp.zeros((1,), jnp.int32), jnp.cumsum(pc)])
    pre = jnp.cumsum(cnt8, axis=0) - cnt8
    off = base[:E][None, :] + pre
    off16 = jnp.pad(off, ((0, 0), (0, 8)))

    block_expert = jnp.clip(
        jnp.searchsorted(base, jnp.arange(NB, dtype=jnp.int32) * BLK,
                         side="right").astype(jnp.int32) - 1, 0, E - 1)

    xs, posA, posB = _sc_scatter(x_flat, i0, i1, lrA, lrB, off16)

    os_rows = _expert_ffn(block_expert, xs, W1, b1, W2, b2)

    y = _sc_combine(os_rows, posA, posB, w0, w1)

    usage_counts = tot.astype(x.dtype)
    usage_fraction = usage_counts / jnp.float32(A)
    zero = jnp.zeros((), dtype=x.dtype)
    return (y.reshape(B, T, D), usage_counts, usage_fraction, zero)
